# baseline (speedup 1.0000x reference)
import functools

import jax
import jax.numpy as jnp
from jax import lax
from jax.experimental import pallas as pl
from jax.experimental.pallas import tpu as pltpu
from jax.experimental.pallas import tpu_sc as plsc

F0, F1, F2 = 5, 6, 2
EMB = 128
E = 320000
NROWS = F0 * F1 * F2

NC, NS = 2, 16
NW = NC * NS
PER_W = E // NW
CHUNK = 80
NCH = PER_W // CHUNK
NBUF = 5


def _sc_body(ab_hbm, w0_hbm, w1_hbm, w2_hbm, out_hbm,
             ab_v, idx_v, rows_v, w0_v, w1_v, w2_v, c_loc, c_sh,
             isem, gsem, osem):
    sid = lax.axis_index("s")
    wid = sid * NC + lax.axis_index("c")
    wbase = wid * PER_W

    @pl.when(sid < 15)
    def _():
        pltpu.sync_copy(w0_hbm, w0_v)
        pltpu.sync_copy(w1_hbm, w1_v)
        pltpu.sync_copy(w2_hbm, w2_v)
        for q in range(4):
            r = sid * 4 + q
            i0 = r // (F1 * F2)
            i1 = (r // F2) % F1
            i2 = r % F2
            for j in range(EMB // 16):
                c_loc[q, pl.ds(j * 16, 16)] = (
                    w0_v[pl.ds(i0 * EMB + j * 16, 16)]
                    + w1_v[pl.ds(i1 * EMB + j * 16, 16)]
                    + w2_v[pl.ds(i2 * EMB + j * 16, 16)]
                )
        pltpu.sync_copy(c_loc, c_sh.at[pl.ds(sid * 4, 4)])
    plsc.subcore_barrier()

    SEG = 3 * CHUNK

    def fire_in(k, b):
        off = (wid * NCH + k) * SEG
        pltpu.async_copy(ab_hbm.at[pl.ds(off, SEG)], ab_v.at[pl.ds(b * SEG, SEG)],
                         isem.at[b])

    def wait_in(k, b):
        off = (wid * NCH + k) * SEG
        pltpu.make_async_copy(ab_hbm.at[pl.ds(off, SEG)],
                              ab_v.at[pl.ds(b * SEG, SEG)], isem.at[b]).wait()

    def wait_out(k, b):
        base = wbase + k * CHUNK
        pltpu.make_async_copy(rows_v.at[b], out_hbm.at[pl.ds(base, CHUNK)],
                              osem.at[b]).wait()

    def step(k, b, fire_next, do_wait_out):
        wait_in(k, b)
        for i in range(CHUNK // 16):
            idx_v[b, pl.ds(i * 16, 16)] = (
                ab_v[pl.ds(b * SEG + i * 16, 16)] * (F1 * F2)
                + ab_v[pl.ds(b * SEG + CHUNK + i * 16, 16)] * F2
                + ab_v[pl.ds(b * SEG + 2 * CHUNK + i * 16, 16)]
            )
        if fire_next:
            fire_in(k + NBUF, b)
        if do_wait_out:
            wait_out(k - NBUF, b)
        pltpu.async_copy(c_sh.at[idx_v.at[b]], rows_v.at[b], gsem.at[b]).wait()
        base = wbase + k * CHUNK
        pltpu.async_copy(rows_v.at[b], out_hbm.at[pl.ds(base, CHUNK)], osem.at[b])

    for b in range(NBUF):
        fire_in(b, b)
    for b in range(NBUF):
        step(b, b, fire_next=True, do_wait_out=False)

    def super_step(g, carry):
        for b in range(NBUF):
            step(g * NBUF + b, b, fire_next=True, do_wait_out=True)
        return carry

    lax.fori_loop(1, NCH // NBUF - 1, super_step, 0)
    for b in range(NBUF):
        step((NCH - NBUF) + b, b, fire_next=False, do_wait_out=True)
    for b in range(NBUF):
        wait_out((NCH - NBUF) + b, b)


@functools.partial(jax.jit, static_argnames=())
def _sc_lookup(ab, w0f, w1f, w2f):
    mesh = plsc.VectorSubcoreMesh(core_axis_name="c", subcore_axis_name="s")
    fn = pl.kernel(
        _sc_body,
        out_type=jax.ShapeDtypeStruct((E, EMB), jnp.float32),
        mesh=mesh,
        scratch_types=[
            pltpu.VMEM((NBUF * 3 * CHUNK,), jnp.int32),
            pltpu.VMEM((NBUF, CHUNK), jnp.int32),
            pltpu.VMEM((NBUF, CHUNK, EMB), jnp.float32),
            pltpu.VMEM((F0 * EMB,), jnp.float32),
            pltpu.VMEM((F1 * EMB,), jnp.float32),
            pltpu.VMEM((F2 * EMB,), jnp.float32),
            pltpu.VMEM((4, EMB), jnp.float32),
            pltpu.VMEM_SHARED((NROWS, EMB), jnp.float32),
            pltpu.SemaphoreType.DMA((NBUF,)),
            pltpu.SemaphoreType.DMA((NBUF,)),
            pltpu.SemaphoreType.DMA((NBUF,)),
        ],
    )
    return fn(ab, w0f, w1f, w2f)


def kernel(edge_attr, W0, W1, W2):
    ea = jnp.asarray(edge_attr, jnp.int32)
    packed = (ea.T.reshape(3, NW, NCH, CHUNK)
              .transpose(1, 2, 0, 3).reshape(3 * E))
    return _sc_lookup(packed, W0.reshape(-1), W1.reshape(-1), W2.reshape(-1))


# speedup vs baseline: 1.1145x; 1.1145x over previous
import functools

import jax
import jax.numpy as jnp
from jax import lax
from jax.experimental import pallas as pl
from jax.experimental.pallas import tpu as pltpu
from jax.experimental.pallas import tpu_sc as plsc

F0, F1, F2 = 5, 6, 2
EMB = 128
E = 320000
NROWS = F0 * F1 * F2

NC, NS = 2, 16
NW = NC * NS
PER_W = E // NW
CHUNK = 80
NCH = PER_W // CHUNK
NBUF = 5


def _sc_body(a0_hbm, a1_hbm, a2_hbm, w0_hbm, w1_hbm, w2_hbm, out_hbm,
             ab_v, idx_v, rows_v, w0_v, w1_v, w2_v, c_loc, c_sh,
             isem, gsem, osem):
    sid = lax.axis_index("s")
    wid = sid * NC + lax.axis_index("c")
    wbase = wid * PER_W

    @pl.when(sid < 15)
    def _():
        pltpu.sync_copy(w0_hbm, w0_v)
        pltpu.sync_copy(w1_hbm, w1_v)
        pltpu.sync_copy(w2_hbm, w2_v)
        for q in range(4):
            r = sid * 4 + q
            i0 = r // (F1 * F2)
            i1 = (r // F2) % F1
            i2 = r % F2
            for j in range(EMB // 16):
                c_loc[q, pl.ds(j * 16, 16)] = (
                    w0_v[pl.ds(i0 * EMB + j * 16, 16)]
                    + w1_v[pl.ds(i1 * EMB + j * 16, 16)]
                    + w2_v[pl.ds(i2 * EMB + j * 16, 16)]
                )
        pltpu.sync_copy(c_loc, c_sh.at[pl.ds(sid * 4, 4)])
    plsc.subcore_barrier()

    SEG = 3 * CHUNK

    def fire_in(k, b):
        base = wbase + k * CHUNK
        s = pl.ds(base, CHUNK)
        pltpu.async_copy(a0_hbm.at[s], ab_v.at[pl.ds(b * SEG, CHUNK)], isem.at[b])
        pltpu.async_copy(a1_hbm.at[s], ab_v.at[pl.ds(b * SEG + CHUNK, CHUNK)],
                         isem.at[b])
        pltpu.async_copy(a2_hbm.at[s], ab_v.at[pl.ds(b * SEG + 2 * CHUNK, CHUNK)],
                         isem.at[b])

    def wait_in(k, b):
        base = wbase + k * CHUNK
        s = pl.ds(base, CHUNK)
        pltpu.make_async_copy(a0_hbm.at[s], ab_v.at[pl.ds(b * SEG, CHUNK)],
                              isem.at[b]).wait()
        pltpu.make_async_copy(a1_hbm.at[s], ab_v.at[pl.ds(b * SEG + CHUNK, CHUNK)],
                              isem.at[b]).wait()
        pltpu.make_async_copy(a2_hbm.at[s],
                              ab_v.at[pl.ds(b * SEG + 2 * CHUNK, CHUNK)],
                              isem.at[b]).wait()

    def wait_out(k, b):
        base = wbase + k * CHUNK
        pltpu.make_async_copy(rows_v.at[b], out_hbm.at[pl.ds(base, CHUNK)],
                              osem.at[b]).wait()

    def step(k, b, fire_next, do_wait_out):
        wait_in(k, b)
        for i in range(CHUNK // 16):
            idx_v[b, pl.ds(i * 16, 16)] = (
                ab_v[pl.ds(b * SEG + i * 16, 16)] * (F1 * F2)
                + ab_v[pl.ds(b * SEG + CHUNK + i * 16, 16)] * F2
                + ab_v[pl.ds(b * SEG + 2 * CHUNK + i * 16, 16)]
            )
        if fire_next:
            fire_in(k + NBUF, b)
        if do_wait_out:
            wait_out(k - NBUF, b)
        pltpu.async_copy(c_sh.at[idx_v.at[b]], rows_v.at[b], gsem.at[b]).wait()
        base = wbase + k * CHUNK
        pltpu.async_copy(rows_v.at[b], out_hbm.at[pl.ds(base, CHUNK)], osem.at[b])

    for b in range(NBUF):
        fire_in(b, b)
    for b in range(NBUF):
        step(b, b, fire_next=True, do_wait_out=False)

    def super_step(g, carry):
        for b in range(NBUF):
            step(g * NBUF + b, b, fire_next=True, do_wait_out=True)
        return carry

    lax.fori_loop(1, NCH // NBUF - 1, super_step, 0)
    for b in range(NBUF):
        step((NCH - NBUF) + b, b, fire_next=False, do_wait_out=True)
    for b in range(NBUF):
        wait_out((NCH - NBUF) + b, b)


@functools.partial(jax.jit, static_argnames=())
def _sc_lookup(a0, a1, a2, w0f, w1f, w2f):
    mesh = plsc.VectorSubcoreMesh(core_axis_name="c", subcore_axis_name="s")
    fn = pl.kernel(
        _sc_body,
        out_type=jax.ShapeDtypeStruct((E, EMB), jnp.float32),
        mesh=mesh,
        scratch_types=[
            pltpu.VMEM((NBUF * 3 * CHUNK,), jnp.int32),
            pltpu.VMEM((NBUF, CHUNK), jnp.int32),
            pltpu.VMEM((NBUF, CHUNK, EMB), jnp.float32),
            pltpu.VMEM((F0 * EMB,), jnp.float32),
            pltpu.VMEM((F1 * EMB,), jnp.float32),
            pltpu.VMEM((F2 * EMB,), jnp.float32),
            pltpu.VMEM((4, EMB), jnp.float32),
            pltpu.VMEM_SHARED((NROWS, EMB), jnp.float32),
            pltpu.SemaphoreType.DMA((NBUF,)),
            pltpu.SemaphoreType.DMA((NBUF,)),
            pltpu.SemaphoreType.DMA((NBUF,)),
        ],
    )
    return fn(a0, a1, a2, w0f, w1f, w2f)


def kernel(edge_attr, W0, W1, W2):
    ea = jnp.asarray(edge_attr, jnp.int32)
    return _sc_lookup(ea[:, 0], ea[:, 1], ea[:, 2],
                      W0.reshape(-1), W1.reshape(-1), W2.reshape(-1))
